# baseline (speedup 1.0000x reference)
import jax
import jax.numpy as jnp
from jax import lax
from jax.experimental import pallas as pl
from jax.experimental.pallas import tpu as pltpu
from jax.experimental.pallas import tpu_sc as plsc

NC = 2
NS = 16
NW = NC * NS
LANES = 16


def _make_sc_call(N, V, MJ, ND, ED, E):
    PER_W = N // NW
    C = 640
    NCHUNK = PER_W // C
    EW = ED - 3

    def body(*refs):
        (idx_hbm, jp_hbm, bb_hbm, emb_hbm, emapt_hbm) = refs[:5]
        ewc_hbm = refs[5:5 + EW]
        btf_hbm, out_hbm = refs[5 + EW], refs[6 + EW]
        (idx_v, jp_v, bb_v, fidx_v, eidx_v, embr_v, eet_v, tail_v,
         btab_v, sem_in, sem_e, sem_m, sem_w, sem_o) = refs[7 + EW:]
        wid = lax.axis_index("s") * NC + lax.axis_index("c")
        lane = lax.iota(jnp.int32, LANES)
        pltpu.sync_copy(btf_hbm, btab_v)

        def base_of(ch):
            return wid * PER_W + ch * C

        def start_in(ch, b):
            base = base_of(ch)
            pltpu.async_copy(idx_hbm.at[pl.ds(base, C)], idx_v[b], sem_in[b])
            pltpu.async_copy(jp_hbm.at[pl.ds(base, C)], jp_v[b], sem_in[b])
            pltpu.async_copy(bb_hbm.at[pl.ds(base, C)], bb_v[b], sem_in[b])

        def wait_in(b):
            for r in (idx_v[b], jp_v[b], bb_v[b]):
                pltpu.make_async_copy(idx_hbm.at[pl.ds(0, C)], r, sem_in[b]).wait()

        def phase_a(ch, b):
            wait_in(b)
            cp_emb = pltpu.async_copy(emb_hbm.at[idx_v[b]], embr_v[b], sem_e[b])

            def fidx_body(i, c2):
                s = pl.ds(i * LANES, LANES)
                fidx_v[b][s] = jp_v[b][s] * V + idx_v[b][s]
                return c2

            lax.fori_loop(0, C // LANES, fidx_body, 0)
            cp_map = pltpu.async_copy(emapt_hbm.at[fidx_v[b]], eidx_v[b], sem_m[b])
            return cp_emb, cp_map

        def phase_b(ch, b, cp_emb, cp_map):
            base = base_of(ch)
            cp_map.wait()

            def eidx_body(i, c2):
                s = pl.ds(i * LANES, LANES)
                eidx_v[b][s] = eidx_v[b][s] + 1
                return c2

            lax.fori_loop(0, C // LANES, eidx_body, 0)
            cps = [pltpu.async_copy(ewc_hbm[c].at[eidx_v[b]], eet_v[b].at[c],
                                    sem_w[b])
                   for c in range(EW)]
            cp_emb.wait()
            cp_oe = pltpu.async_copy(
                embr_v[b], out_hbm.at[pl.ds(base, C), pl.ds(0, ND)], sem_o[b])
            for cp in cps:
                cp.wait()

            def tr_body(i, c2):
                t16 = lane + i * LANES
                s = pl.ds(i * LANES, LANES)
                for c in range(EW):
                    plsc.store_scatter(
                        tail_v[b], [t16, jnp.full((LANES,), c, jnp.int32)],
                        eet_v[b][c, s])
                bb16 = bb_v[b][s]
                for j in range(3):
                    plsc.store_scatter(
                        tail_v[b], [t16, jnp.full((LANES,), EW + j, jnp.int32)],
                        plsc.load_gather(btab_v, [bb16 * 3 + j]))
                return c2

            lax.fori_loop(0, C // LANES, tr_body, 0)
            cp_ot = pltpu.async_copy(
                tail_v[b], out_hbm.at[pl.ds(base, C), pl.ds(ND, ED)], sem_o[b])
            return cp_oe, cp_ot

        start_in(0, 0)
        start_in(1, 1)
        inflight_a = phase_a(0, 0)
        inflight_o = [None, None]
        for ch in range(NCHUNK):
            b = ch % 2
            nxt = (ch + 1) % 2
            a_next = None
            if ch + 1 < NCHUNK:
                if inflight_o[nxt] is not None:
                    for cp in inflight_o[nxt]:
                        cp.wait()
                    inflight_o[nxt] = None
                a_next = phase_a(ch + 1, nxt)
            if inflight_o[b] is not None:
                for cp in inflight_o[b]:
                    cp.wait()
            inflight_o[b] = phase_b(ch, b, *inflight_a)
            inflight_a = a_next
            if ch + 2 < NCHUNK:
                start_in(ch + 2, b)
        for cps in inflight_o:
            if cps is not None:
                for cp in cps:
                    cp.wait()

    D = ND + ED
    dbl = lambda shape, dt: [pltpu.VMEM(shape, dt), pltpu.VMEM(shape, dt)]
    sem2 = lambda: [pltpu.SemaphoreType.DMA, pltpu.SemaphoreType.DMA]
    return pl.kernel(
        body,
        out_type=jax.ShapeDtypeStruct((N, D), jnp.float32),
        mesh=plsc.VectorSubcoreMesh(core_axis_name="c", subcore_axis_name="s",
                                    num_cores=NC, num_subcores=NS),
        compiler_params=pltpu.CompilerParams(use_tc_tiling_on_sc=False,
                                             needs_layout_passes=False),
        scratch_types=[
            dbl((C,), jnp.int32),
            dbl((C,), jnp.int32),
            dbl((C,), jnp.int32),
            dbl((C,), jnp.int32),
            dbl((C,), jnp.int32),
            dbl((C, ND), jnp.float32),
            dbl((EW, C), jnp.float32),
            dbl((C, ED), jnp.float32),
            pltpu.VMEM((12,), jnp.float32),
            sem2(),
            sem2(),
            sem2(),
            sem2(),
            sem2(),
        ],
    )


def kernel(idx, joint_info, embedding, edge_idx_map, edge_emb_weight, bond_type):
    B, L = idx.shape
    N = B * L
    V, ND = embedding.shape
    MJ = edge_idx_map.shape[1]
    E, EW = edge_emb_weight.shape
    ED = EW + 3
    idx_f = idx.reshape(N)
    jp_f = joint_info[..., 0].reshape(N)
    bb_f = joint_info[..., 1].reshape(N)
    emap_t = edge_idx_map.T.reshape(MJ * V)
    ewt = edge_emb_weight.T
    ew_cols = [ewt[c] for c in range(EW)]
    bt_f = bond_type.reshape(-1)
    out = _make_sc_call(N, V, MJ, ND, ED, E)(
        idx_f, jp_f, bb_f, embedding, emap_t, *ew_cols, bt_f)
    return out.reshape(B, L, ND + ED)
